# baseline (speedup 1.0000x reference)
import jax
import jax.numpy as jnp
from jax import lax
from jax.experimental import pallas as pl
from jax.experimental.pallas import tpu as pltpu
from jax.experimental.pallas import tpu_sc as plsc

D_MODEL = 768
MAX_REL = 128
VOCAB = 2 * MAX_REL + 1
S = 512

NC = 2
NS = 16
NW = NC * NS
CHUNK = 64
NCHUNK = S // CHUNK

EPAD = S - MAX_REL
EXT_ROWS = 2 * EPAD + VOCAB
EXT8_ROWS = EXT_ROWS + 7
KROWS = 16
WINDOW = CHUNK + 8 * (KROWS - 1)
DSLICE = D_MODEL // 3
NSTEPS = 3 * NCHUNK


def _rpe_body(ext8_hbm, out_hbm, buf0, buf1, sem0, sem1):
    wid = lax.axis_index("s") * NC + lax.axis_index("c")
    blk = wid // 8
    ph = wid % 8
    bufs = (buf0, buf1)
    sems = (sem0, sem1)

    def src_of(t):
        d = t % 3
        c = t // 3
        off = pl.multiple_of(
            ph * EXT8_ROWS + (EPAD + 8) - blk * 128 + c * CHUNK, 8)
        return ext8_hbm.at[pl.ds(off, WINDOW), pl.ds(d * DSLICE, DSLICE)]

    def fire(t, b):
        pltpu.async_copy(src_of(t), bufs[b], sems[b])

    def drain_write(t, b):
        d = t % 3
        c = t // 3
        j0 = c * CHUNK
        pltpu.make_async_copy(src_of(t), bufs[b], sems[b]).wait()
        for k in range(KROWS):
            i = blk * 128 + ph + 8 * k
            pltpu.sync_copy(bufs[b].at[pl.ds(8 * (KROWS - 1 - k), CHUNK)],
                            out_hbm.at[pl.ds(i * S + j0, CHUNK),
                                       pl.ds(d * DSLICE, DSLICE)])

    fire(0, 0)

    def outer(u, carry):
        t0 = 2 * u
        fire(t0 + 1, 1)
        drain_write(t0, 0)

        @pl.when(t0 + 2 < NSTEPS)
        def _():
            fire(t0 + 2, 0)

        drain_write(t0 + 1, 1)
        return carry

    lax.fori_loop(0, NSTEPS // 2, outer, 0)


def kernel(seq_len, table):
    ext = jnp.concatenate([
        jnp.broadcast_to(table[0:1], (EPAD, D_MODEL)),
        table,
        jnp.broadcast_to(table[VOCAB - 1:VOCAB], (EPAD, D_MODEL)),
    ], axis=0)
    ext8 = jnp.concatenate(
        [jnp.pad(ext, ((p, 7 - p), (0, 0))) for p in range(8)], axis=0)

    out = pl.kernel(
        _rpe_body,
        mesh=plsc.VectorSubcoreMesh(core_axis_name="c", subcore_axis_name="s"),
        out_type=jax.ShapeDtypeStruct((S * S, D_MODEL), jnp.float32),
        scratch_types=[
            pltpu.VMEM((WINDOW, DSLICE), jnp.float32),
            pltpu.VMEM((WINDOW, DSLICE), jnp.float32),
            pltpu.SemaphoreType.DMA,
            pltpu.SemaphoreType.DMA,
        ],
    )(ext8)
    return out.reshape(S, S, D_MODEL)


# speedup vs baseline: 1.0529x; 1.0529x over previous
import jax
import jax.numpy as jnp
from jax import lax
from jax.experimental import pallas as pl
from jax.experimental.pallas import tpu as pltpu
from jax.experimental.pallas import tpu_sc as plsc

D_MODEL = 768
MAX_REL = 128
VOCAB = 2 * MAX_REL + 1
S = 512

NC = 2
NS = 16
NW = NC * NS
CHUNK = 64
NCHUNK = S // CHUNK

EPAD = S - MAX_REL
EXT_ROWS = 2 * EPAD + VOCAB
EXT8_ROWS = EXT_ROWS + 7
KROWS = 16
WINDOW = CHUNK + 8 * (KROWS - 1)
DHALF = D_MODEL // 2


def _rpe_body(ext8_hbm, out_hbm, buf):
    wid = lax.axis_index("s") * NC + lax.axis_index("c")
    blk = wid // 8
    ph = wid % 8

    def step(t, carry):
        d = t & 1
        c = t >> 1
        j0 = c * CHUNK
        i_base = blk * 128 + ph
        off = pl.multiple_of(
            ph * EXT8_ROWS + (EPAD + 8) - blk * 128 + j0, 8)
        pltpu.sync_copy(ext8_hbm.at[pl.ds(off, WINDOW), pl.ds(d * DHALF, DHALF)],
                        buf.at[pl.ds(0, WINDOW)])
        for k in range(KROWS):
            i = i_base + 8 * k
            pltpu.sync_copy(buf.at[pl.ds(8 * (KROWS - 1 - k), CHUNK)],
                            out_hbm.at[pl.ds(i * S + j0, CHUNK),
                                       pl.ds(d * DHALF, DHALF)])
        return carry

    lax.fori_loop(0, 2 * NCHUNK, step, 0)


def kernel(seq_len, table):
    ext = jnp.concatenate([
        jnp.broadcast_to(table[0:1], (EPAD, D_MODEL)),
        table,
        jnp.broadcast_to(table[VOCAB - 1:VOCAB], (EPAD, D_MODEL)),
    ], axis=0)
    ext8 = jnp.concatenate(
        [jnp.pad(ext, ((p, 7 - p), (0, 0))) for p in range(8)], axis=0)

    out = pl.kernel(
        _rpe_body,
        mesh=plsc.VectorSubcoreMesh(core_axis_name="c", subcore_axis_name="s"),
        out_type=jax.ShapeDtypeStruct((S * S, D_MODEL), jnp.float32),
        scratch_types=[
            pltpu.VMEM((WINDOW, DHALF), jnp.float32),
        ],
    )(ext8)
    return out.reshape(S, S, D_MODEL)


# speedup vs baseline: 1.1040x; 1.0485x over previous
import jax
import jax.numpy as jnp
from jax import lax
from jax.experimental import pallas as pl
from jax.experimental.pallas import tpu as pltpu
from jax.experimental.pallas import tpu_sc as plsc

D_MODEL = 768
MAX_REL = 128
VOCAB = 2 * MAX_REL + 1
S = 512

NC = 2
NS = 16
NW = NC * NS
CHUNK = 128
NCHUNK = S // CHUNK

EPAD = S - MAX_REL
EXT_ROWS = 2 * EPAD + VOCAB
EXT8_ROWS = EXT_ROWS + 7
KROWS = 16
WINDOW = CHUNK + 8 * (KROWS - 1)
DHALF = D_MODEL // 2


def _rpe_body(ext8_hbm, out_hbm, buf):
    wid = lax.axis_index("s") * NC + lax.axis_index("c")
    blk = wid // 8
    ph = wid % 8

    def step(t, carry):
        d = t & 1
        c = t >> 1
        j0 = c * CHUNK
        i_base = blk * 128 + ph
        off = pl.multiple_of(
            ph * EXT8_ROWS + (EPAD + 8) - blk * 128 + j0, 8)
        pltpu.sync_copy(ext8_hbm.at[pl.ds(off, WINDOW), pl.ds(d * DHALF, DHALF)],
                        buf.at[pl.ds(0, WINDOW)])
        for k in range(KROWS):
            i = i_base + 8 * k
            pltpu.sync_copy(buf.at[pl.ds(8 * (KROWS - 1 - k), CHUNK)],
                            out_hbm.at[pl.ds(i * S + j0, CHUNK),
                                       pl.ds(d * DHALF, DHALF)])
        return carry

    lax.fori_loop(0, 2 * NCHUNK, step, 0)


def kernel(seq_len, table):
    ext = jnp.concatenate([
        jnp.broadcast_to(table[0:1], (EPAD, D_MODEL)),
        table,
        jnp.broadcast_to(table[VOCAB - 1:VOCAB], (EPAD, D_MODEL)),
    ], axis=0)
    ext8 = jnp.concatenate(
        [jnp.pad(ext, ((p, 7 - p), (0, 0))) for p in range(8)], axis=0)

    out = pl.kernel(
        _rpe_body,
        mesh=plsc.VectorSubcoreMesh(core_axis_name="c", subcore_axis_name="s"),
        out_type=jax.ShapeDtypeStruct((S * S, D_MODEL), jnp.float32),
        scratch_types=[
            pltpu.VMEM((WINDOW, DHALF), jnp.float32),
        ],
    )(ext8)
    return out.reshape(S, S, D_MODEL)


# speedup vs baseline: 1.1350x; 1.0281x over previous
import jax
import jax.numpy as jnp
from jax import lax
from jax.experimental import pallas as pl
from jax.experimental.pallas import tpu as pltpu
from jax.experimental.pallas import tpu_sc as plsc

D_MODEL = 768
MAX_REL = 128
VOCAB = 2 * MAX_REL + 1
S = 512

NC = 2
NS = 16
NW = NC * NS
CHUNK = 256
NCHUNK = S // CHUNK

EPAD = S - MAX_REL
EXT_ROWS = 2 * EPAD + VOCAB
EXT8_ROWS = EXT_ROWS + 7
KROWS = 16
WINDOW = CHUNK + 8 * (KROWS - 1)
DHALF = D_MODEL // 3


def _rpe_body(ext8_hbm, out_hbm, buf):
    wid = lax.axis_index("s") * NC + lax.axis_index("c")
    blk = wid // 8
    ph = wid % 8

    def step(t, carry):
        d = t % 3
        c = t // 3
        j0 = c * CHUNK
        i_base = blk * 128 + ph
        off = pl.multiple_of(
            ph * EXT8_ROWS + (EPAD + 8) - blk * 128 + j0, 8)
        pltpu.sync_copy(ext8_hbm.at[pl.ds(off, WINDOW), pl.ds(d * DHALF, DHALF)],
                        buf.at[pl.ds(0, WINDOW)])
        for k in range(KROWS):
            i = i_base + 8 * k
            pltpu.sync_copy(buf.at[pl.ds(8 * (KROWS - 1 - k), CHUNK)],
                            out_hbm.at[pl.ds(i * S + j0, CHUNK),
                                       pl.ds(d * DHALF, DHALF)])
        return carry

    lax.fori_loop(0, 3 * NCHUNK, step, 0)


def kernel(seq_len, table):
    ext = jnp.concatenate([
        jnp.broadcast_to(table[0:1], (EPAD, D_MODEL)),
        table,
        jnp.broadcast_to(table[VOCAB - 1:VOCAB], (EPAD, D_MODEL)),
    ], axis=0)
    ext8 = jnp.concatenate(
        [jnp.pad(ext, ((p, 7 - p), (0, 0))) for p in range(8)], axis=0)

    out = pl.kernel(
        _rpe_body,
        mesh=plsc.VectorSubcoreMesh(core_axis_name="c", subcore_axis_name="s"),
        out_type=jax.ShapeDtypeStruct((S * S, D_MODEL), jnp.float32),
        scratch_types=[
            pltpu.VMEM((WINDOW, DHALF), jnp.float32),
        ],
    )(ext8)
    return out.reshape(S, S, D_MODEL)
